# baseline (device time: 28796 ns/iter reference)
import jax
import jax.numpy as jnp
from jax import lax
from jax.experimental import pallas as pl
from jax.experimental.pallas import tpu as pltpu

N_CHUNKS = 4


def kernel(x, pi):
    s, m, n = x.shape
    rows = m // N_CHUNKS
    x = pltpu.with_memory_space_constraint(x, pltpu.MemorySpace.HBM)
    pi = pltpu.with_memory_space_constraint(pi, pltpu.MemorySpace.HBM)

    def body(
        x_ref,
        pi_ref,
        out_ref,
        in_stage,
        wire_stage,
        pi_smem,
        in_sems,
        pi_sem,
        send_sems,
        recv_sems,
    ):
        my_x = lax.axis_index("x")
        my_y = lax.axis_index("y")
        other_y = 1 - my_y

        pi_copy = pltpu.make_async_copy(pi_ref, pi_smem, pi_sem)
        pi_copy.start()
        in_copies = []
        for c in range(N_CHUNKS):
            sl = pl.ds(c * rows, rows)
            cp = pltpu.make_async_copy(
                x_ref.at[0, sl], in_stage.at[c % 2], in_sems.at[c % 2]
            )
            in_copies.append(cp)
        in_copies[0].start()
        in_copies[1].start()

        barrier_sem = pltpu.get_barrier_semaphore()
        pl.semaphore_signal(
            barrier_sem,
            inc=1,
            device_id=(my_x, other_y),
            device_id_type=pl.DeviceIdType.MESH,
        )
        pl.semaphore_wait(barrier_sem, 1)

        pi_copy.wait()
        dst_y = pi_smem[my_y]

        rdmas = []
        for c in range(N_CHUNKS):
            sl = pl.ds(c * rows, rows)
            in_copies[c].wait()
            wire_stage[sl, :] = in_stage[c % 2].astype(jnp.bfloat16)
            if c + 2 < N_CHUNKS:
                in_copies[c + 2].start()
            rdma = pltpu.make_async_remote_copy(
                src_ref=wire_stage.at[sl],
                dst_ref=out_ref.at[0, sl],
                send_sem=send_sems.at[c],
                recv_sem=recv_sems.at[c],
                device_id=(my_x, dst_y),
                device_id_type=pl.DeviceIdType.MESH,
            )
            rdma.start()
            rdmas.append(rdma)

        for c in range(N_CHUNKS):
            rdmas[c].wait_recv()
        for c in range(N_CHUNKS):
            rdmas[c].wait_send()

    return pl.pallas_call(
        body,
        out_shape=jax.ShapeDtypeStruct((s, m, n), jnp.bfloat16),
        in_specs=[
            pl.BlockSpec(memory_space=pltpu.MemorySpace.HBM),
            pl.BlockSpec(memory_space=pltpu.MemorySpace.HBM),
        ],
        out_specs=pl.BlockSpec(memory_space=pltpu.MemorySpace.HBM),
        scratch_shapes=[
            pltpu.VMEM((2, rows, n), jnp.float32),
            pltpu.VMEM((m, n), jnp.bfloat16),
            pltpu.SMEM((2,), jnp.int32),
            pltpu.SemaphoreType.DMA((2,)),
            pltpu.SemaphoreType.DMA,
            pltpu.SemaphoreType.DMA((N_CHUNKS,)),
            pltpu.SemaphoreType.DMA((N_CHUNKS,)),
        ],
        compiler_params=pltpu.CompilerParams(collective_id=0),
    )(x, pi)


# device time: 21126 ns/iter; 1.3631x vs baseline; 1.3631x over previous
import jax
import jax.numpy as jnp
from jax import lax
from jax.experimental import pallas as pl
from jax.experimental.pallas import tpu as pltpu

N_CHUNKS = 8


def kernel(x, pi):
    s, m, n = x.shape
    half = m // 2
    hc = half // N_CHUNKS
    x = pltpu.with_memory_space_constraint(x, pltpu.MemorySpace.HBM)
    pi = pltpu.with_memory_space_constraint(pi, pltpu.MemorySpace.HBM)

    def body(
        x_ref,
        pi_ref,
        out_ref,
        in_stage,
        wire_stage,
        yrecv,
        pi_smem,
        in_sems,
        pi_sem,
        cp_sems,
        ysend_sems,
        yrecv_sems,
        xsend_sems,
        xrecv_sems,
    ):
        my_x = lax.axis_index("x")
        my_y = lax.axis_index("y")
        other_x = 1 - my_x
        other_y = 1 - my_y
        base = my_x * half

        pi_copy = pltpu.make_async_copy(pi_ref, pi_smem, pi_sem)
        pi_copy.start()
        in_copies = []
        for c in range(N_CHUNKS):
            cp = pltpu.make_async_copy(
                x_ref.at[0, pl.ds(base + c * hc, hc)],
                in_stage.at[c % 2],
                in_sems.at[c % 2],
            )
            in_copies.append(cp)
        in_copies[0].start()
        in_copies[1].start()

        barrier_sem = pltpu.get_barrier_semaphore()
        for nbr in [(my_x, other_y), (other_x, my_y)]:
            pl.semaphore_signal(
                barrier_sem,
                inc=1,
                device_id=nbr,
                device_id_type=pl.DeviceIdType.MESH,
            )
        pl.semaphore_wait(barrier_sem, 2)

        pi_copy.wait()
        dst_y = pi_smem[my_y]

        y_rdmas = []
        for c in range(N_CHUNKS):
            sl = pl.ds(c * hc, hc)
            in_copies[c].wait()
            wire_stage[sl, :] = in_stage[c % 2].astype(jnp.bfloat16)
            if c + 2 < N_CHUNKS:
                in_copies[c + 2].start()
            rdma = pltpu.make_async_remote_copy(
                src_ref=wire_stage.at[sl],
                dst_ref=yrecv.at[sl],
                send_sem=ysend_sems.at[c],
                recv_sem=yrecv_sems.at[c],
                device_id=(my_x, dst_y),
                device_id_type=pl.DeviceIdType.MESH,
            )
            rdma.start()
            y_rdmas.append(rdma)

        x_rdmas = []
        local_cps = []
        for c in range(N_CHUNKS):
            sl = pl.ds(c * hc, hc)
            osl = pl.ds(base + c * hc, hc)
            y_rdmas[c].wait_recv()
            fwd = pltpu.make_async_remote_copy(
                src_ref=yrecv.at[sl],
                dst_ref=out_ref.at[0, osl],
                send_sem=xsend_sems.at[c],
                recv_sem=xrecv_sems.at[c],
                device_id=(other_x, my_y),
                device_id_type=pl.DeviceIdType.MESH,
            )
            fwd.start()
            x_rdmas.append(fwd)
            cp = pltpu.make_async_copy(yrecv.at[sl], out_ref.at[0, osl], cp_sems.at[c])
            cp.start()
            local_cps.append(cp)

        for c in range(N_CHUNKS):
            x_rdmas[c].wait_recv()
        for c in range(N_CHUNKS):
            local_cps[c].wait()
            y_rdmas[c].wait_send()
            x_rdmas[c].wait_send()

    return pl.pallas_call(
        body,
        out_shape=jax.ShapeDtypeStruct((s, m, n), jnp.bfloat16),
        in_specs=[
            pl.BlockSpec(memory_space=pltpu.MemorySpace.HBM),
            pl.BlockSpec(memory_space=pltpu.MemorySpace.HBM),
        ],
        out_specs=pl.BlockSpec(memory_space=pltpu.MemorySpace.HBM),
        scratch_shapes=[
            pltpu.VMEM((2, hc, n), jnp.float32),
            pltpu.VMEM((half, n), jnp.bfloat16),
            pltpu.VMEM((half, n), jnp.bfloat16),
            pltpu.SMEM((2,), jnp.int32),
            pltpu.SemaphoreType.DMA((2,)),
            pltpu.SemaphoreType.DMA,
            pltpu.SemaphoreType.DMA((N_CHUNKS,)),
            pltpu.SemaphoreType.DMA((N_CHUNKS,)),
            pltpu.SemaphoreType.DMA((N_CHUNKS,)),
            pltpu.SemaphoreType.DMA((N_CHUNKS,)),
            pltpu.SemaphoreType.DMA((N_CHUNKS,)),
        ],
        compiler_params=pltpu.CompilerParams(collective_id=0),
    )(x, pi)
